# baseline (speedup 1.0000x reference)
import jax
import jax.numpy as jnp
from jax import lax
from jax.experimental import pallas as pl
from jax.experimental.pallas import tpu as pltpu

_EPS = 1e-06
_HB = 128


def _ln_body(bits_ref, x_ref, w_ref, b_ref, o_ref):
    bidx = pl.program_id(0)
    hb = pl.program_id(1)
    x = x_ref[...]
    u = jnp.mean(x, axis=2, keepdims=True)
    s2 = jnp.mean(x * x, axis=2, keepdims=True)
    v = s2 - u * u
    r = lax.rsqrt(v + _EPS)
    wt = jnp.transpose(w_ref[...])
    bt = jnp.transpose(b_ref[...])
    ln = (x - u) * (r * wt) + bt
    wc = jnp.right_shift(lax.broadcasted_iota(jnp.int32, (1, 1, 1, 128), 3), 3)
    for k in range(_HB // 8):
        s = bits_ref[bidx, hb * (_HB // 8) + k]
        bit = jnp.right_shift(s, wc) & 1
        o_ref[0, 8 * k:8 * k + 8] = jnp.where(
            bit[0] != 0, ln[0, 8 * k:8 * k + 8], 0.0)


def kernel(x, active, weight, bias):
    B, H, W, C = x.shape
    bits = jnp.sum(active[:, 0].astype(jnp.int32) << jnp.arange(16, dtype=jnp.int32),
                   axis=-1, dtype=jnp.int32)
    xt = jnp.transpose(x, (0, 1, 3, 2))
    out_t = pl.pallas_call(
        _ln_body,
        grid=(B, H // _HB),
        in_specs=[
            pl.BlockSpec(memory_space=pltpu.SMEM),
            pl.BlockSpec((1, _HB, C, W), lambda b, h: (b, h, 0, 0)),
            pl.BlockSpec((1, C), lambda b, h: (0, 0)),
            pl.BlockSpec((1, C), lambda b, h: (0, 0)),
        ],
        out_specs=pl.BlockSpec((1, _HB, C, W), lambda b, h: (b, h, 0, 0)),
        out_shape=jax.ShapeDtypeStruct((B, H, C, W), x.dtype),
        compiler_params=pltpu.CompilerParams(
            dimension_semantics=("parallel", "parallel")),
    )(bits, xt, weight.reshape(1, C), bias.reshape(1, C))
    return jnp.transpose(out_t, (0, 1, 3, 2))


# speedup vs baseline: 1.1403x; 1.1403x over previous
import jax
import jax.numpy as jnp
from jax import lax
from jax.experimental import pallas as pl
from jax.experimental.pallas import tpu as pltpu

_EPS = 1e-06
_HB = 128


def _ln_body(bits_ref, x_ref, w_ref, b_ref, o_ref):
    bidx = pl.program_id(0)
    hb = pl.program_id(1)
    x = x_ref[...]
    u = jnp.mean(x, axis=2, keepdims=True)
    s2 = jnp.mean(x * x, axis=2, keepdims=True)
    v = s2 - u * u
    r = lax.rsqrt(v + _EPS)
    ln = (x - u) * (r * w_ref[...]) + b_ref[...]
    wc = jnp.right_shift(lax.broadcasted_iota(jnp.int32, (1, 1, 1, 128), 3), 3)
    for k in range(_HB // 8):
        s = bits_ref[bidx, hb * (_HB // 8) + k]
        bit = jnp.right_shift(s, wc) & 1
        o_ref[0, 8 * k:8 * k + 8] = jnp.where(
            bit[0] != 0, ln[0, 8 * k:8 * k + 8], 0.0)


def kernel(x, active, weight, bias):
    B, H, W, C = x.shape
    bits = jnp.sum(active[:, 0].astype(jnp.int32) << jnp.arange(16, dtype=jnp.int32),
                   axis=-1, dtype=jnp.int32)
    w_bc = jnp.broadcast_to(weight[:, None], (C, W))
    b_bc = jnp.broadcast_to(bias[:, None], (C, W))
    xt = jnp.transpose(x, (0, 1, 3, 2))
    out_t = pl.pallas_call(
        _ln_body,
        grid=(B, H // _HB),
        in_specs=[
            pl.BlockSpec(memory_space=pltpu.SMEM),
            pl.BlockSpec((1, _HB, C, W), lambda b, h: (b, h, 0, 0)),
            pl.BlockSpec((C, W), lambda b, h: (0, 0)),
            pl.BlockSpec((C, W), lambda b, h: (0, 0)),
        ],
        out_specs=pl.BlockSpec((1, _HB, C, W), lambda b, h: (b, h, 0, 0)),
        out_shape=jax.ShapeDtypeStruct((B, H, C, W), x.dtype),
        compiler_params=pltpu.CompilerParams(
            dimension_semantics=("parallel", "parallel")),
    )(bits, xt, w_bc, b_bc)
    return jnp.transpose(out_t, (0, 1, 3, 2))


# speedup vs baseline: 1.1509x; 1.0093x over previous
import jax
import jax.numpy as jnp
from jax import lax
from jax.experimental import pallas as pl
from jax.experimental.pallas import tpu as pltpu

_EPS = 1e-06
_HB = 128


def _ln_body(bits_ref, x_ref, wb_ref, o_ref):
    bidx = pl.program_id(0)
    hb = pl.program_id(1)
    x = x_ref[...]
    u = jnp.mean(x, axis=2, keepdims=True)
    s2 = jnp.mean(x * x, axis=2, keepdims=True)
    v = s2 - u * u
    r = lax.rsqrt(v + _EPS)
    ln = (x - u) * (r * wb_ref[0]) + wb_ref[1]
    wc = jnp.right_shift(lax.broadcasted_iota(jnp.int32, (1, 1, 1, 128), 3), 3)
    for k in range(_HB // 8):
        s = bits_ref[bidx, hb * (_HB // 8) + k]
        bit = jnp.right_shift(s, wc) & 1
        o_ref[0, 8 * k:8 * k + 8] = jnp.where(
            bit[0] != 0, ln[0, 8 * k:8 * k + 8], 0.0)


def kernel(x, active, weight, bias):
    B, H, W, C = x.shape
    bits = jnp.sum(active[:, 0].astype(jnp.int32) << jnp.arange(16, dtype=jnp.int32),
                   axis=-1, dtype=jnp.int32)
    wb = jnp.broadcast_to(jnp.stack([weight, bias])[:, :, None], (2, C, W))
    xt = jnp.transpose(x, (0, 1, 3, 2))
    out_t = pl.pallas_call(
        _ln_body,
        grid=(B, H // _HB),
        in_specs=[
            pl.BlockSpec(memory_space=pltpu.SMEM),
            pl.BlockSpec((1, _HB, C, W), lambda b, h: (b, h, 0, 0)),
            pl.BlockSpec((2, C, W), lambda b, h: (0, 0, 0)),
        ],
        out_specs=pl.BlockSpec((1, _HB, C, W), lambda b, h: (b, h, 0, 0)),
        out_shape=jax.ShapeDtypeStruct((B, H, C, W), x.dtype),
        compiler_params=pltpu.CompilerParams(
            dimension_semantics=("parallel", "parallel")),
    )(bits, xt, wb)
    return jnp.transpose(out_t, (0, 1, 3, 2))


# speedup vs baseline: 1.1554x; 1.0039x over previous
import jax
import jax.numpy as jnp
from jax import lax
from jax.experimental import pallas as pl
from jax.experimental.pallas import tpu as pltpu

_EPS = 1e-06
_HB = 128
_NB = 2


def _ln_body(bits_ref, x_ref, wb_ref, o_ref):
    bidx = pl.program_id(0)
    hb = pl.program_id(1)
    x = x_ref[...]
    u = jnp.mean(x, axis=2, keepdims=True)
    s2 = jnp.mean(x * x, axis=2, keepdims=True)
    v = s2 - u * u
    r = lax.rsqrt(v + _EPS)
    ln = (x - u) * (r * wb_ref[0]) + wb_ref[1]
    wc = jnp.right_shift(lax.broadcasted_iota(jnp.int32, (1, 1, 1, 128), 3), 3)
    nb = x.shape[0]
    for i in range(nb):
        for k in range(_HB // 8):
            s = bits_ref[nb * bidx + i, hb * (_HB // 8) + k]
            bit = jnp.right_shift(s, wc) & 1
            o_ref[i, 8 * k:8 * k + 8] = jnp.where(
                bit[0] != 0, ln[i, 8 * k:8 * k + 8], 0.0)


def kernel(x, active, weight, bias):
    B, H, W, C = x.shape
    bits = jnp.sum(active[:, 0].astype(jnp.int32) << jnp.arange(16, dtype=jnp.int32),
                   axis=-1, dtype=jnp.int32)
    wb = jnp.broadcast_to(jnp.stack([weight, bias])[:, :, None], (2, C, W))
    xt = jnp.transpose(x, (0, 1, 3, 2))
    out_t = pl.pallas_call(
        _ln_body,
        grid=(B // _NB, H // _HB),
        in_specs=[
            pl.BlockSpec(memory_space=pltpu.SMEM),
            pl.BlockSpec((_NB, _HB, C, W), lambda b, h: (b, h, 0, 0)),
            pl.BlockSpec((2, C, W), lambda b, h: (0, 0, 0)),
        ],
        out_specs=pl.BlockSpec((_NB, _HB, C, W), lambda b, h: (b, h, 0, 0)),
        out_shape=jax.ShapeDtypeStruct((B, H, C, W), x.dtype),
        compiler_params=pltpu.CompilerParams(
            dimension_semantics=("parallel", "parallel")),
    )(bits, xt, wb)
    return jnp.transpose(out_t, (0, 1, 3, 2))
